# Initial kernel scaffold; baseline (speedup 1.0000x reference)
import jax
import jax.numpy as jnp
from jax.experimental import pallas as pl


def kernel(node_reps, edges, edge_weights, prep_gamma, prep_beta, prep_mean, prep_var, prep_W, prep_b, upd_gamma, upd_beta, upd_mean, upd_var, upd_W, upd_b):
    raise NotImplementedError("write your pallas kernel here")


# speedup vs baseline: 2.8304x; 2.8304x over previous
import functools
import jax
import jax.numpy as jnp
from jax import lax
from jax.experimental import pallas as pl
from jax.experimental.pallas import tpu as pltpu
import jax.experimental.pallas.tpu_sc as plsc

N_NODES = 10000
N_EDGES = 160000
D = 256
H = 64
NS = 16
EPT = N_EDGES // NS
K = 80
NCHUNK = EPT // K
NPAD = 10240
STRIPE = NPAD // NS
OSUB = 5
OROWS = STRIPE // OSUB
BN = 1000


def _mm1_body(x_ref, w_ref, b_ref, p0_ref, p1_ref, p2_ref, p3_ref):
    h = jnp.dot(x_ref[...], w_ref[...], preferred_element_type=jnp.float32)
    h = jnp.maximum(h + b_ref[...], 0.0)
    p0_ref[...] = h[:, 0 * H:1 * H]
    p1_ref[...] = h[:, 1 * H:2 * H]
    p2_ref[...] = h[:, 2 * H:3 * H]
    p3_ref[...] = h[:, 3 * H:4 * H]


def _mm2_body(x_ref, a0_ref, a1_ref, a2_ref, a3_ref, wa_ref, wb0_ref,
              wb1_ref, wb2_ref, wb3_ref, c_ref, o_ref):
    acc = jnp.dot(x_ref[...], wa_ref[...], preferred_element_type=jnp.float32)
    acc += jnp.dot(a0_ref[...], wb0_ref[...], preferred_element_type=jnp.float32)
    acc += jnp.dot(a1_ref[...], wb1_ref[...], preferred_element_type=jnp.float32)
    acc += jnp.dot(a2_ref[...], wb2_ref[...], preferred_element_type=jnp.float32)
    acc += jnp.dot(a3_ref[...], wb3_ref[...], preferred_element_type=jnp.float32)
    o_ref[...] = jnp.maximum(acc + c_ref[...], 0.0)


def _sc_body(pA, pB, nbr, dst, w, zrow, zcnt, ones_h,
             aggA_out, aggB_out,
             nbr_v, w_v, dst_v, rows_v, ones_v, obuf_v, cnt_v,
             agg_sh, cnt_sh, sem):
    c = lax.axis_index("c")
    s = lax.axis_index("s")
    base_n = s * STRIPE

    pltpu.sync_copy(zrow, agg_sh.at[pl.ds(base_n, STRIPE)])
    pltpu.sync_copy(zcnt, cnt_sh.at[pl.ds(base_n, STRIPE)])
    pltpu.sync_copy(nbr.at[pl.ds(s * EPT, EPT)], nbr_v)
    pltpu.sync_copy(w.at[pl.ds(s * EPT, EPT)], w_v)
    pltpu.sync_copy(dst.at[s], dst_v)
    pltpu.sync_copy(ones_h, ones_v)
    plsc.subcore_barrier()

    def chunk_body(g, carry):
        idx = nbr_v.at[pl.ds(g * K, K)]
        @pl.when(c == 0)
        def _():
            pltpu.async_copy(pA.at[idx], rows_v, sem).wait()
        @pl.when(c == 1)
        def _():
            pltpu.async_copy(pB.at[idx], rows_v, sem).wait()

        def edge_body(e, carry2):
            wv = plsc.load_gather(w_v, [jnp.full((16,), g * K + e, jnp.int32)])
            for j in range(H // 16):
                sl = pl.ds(j * 16, 16)
                rows_v[e, sl] = rows_v[e, sl] * wv
            return carry2
        lax.fori_loop(0, K, edge_body, 0, unroll=2)

        didx = dst_v.at[g]
        pltpu.sync_copy(rows_v, agg_sh.at[didx], add=True)
        pltpu.sync_copy(ones_v, cnt_sh.at[didx], add=True)
        return carry
    lax.fori_loop(0, NCHUNK, chunk_body, 0)
    plsc.subcore_barrier()

    def out_piece(t, carry):
        row0 = base_n + t * OROWS
        pltpu.sync_copy(agg_sh.at[pl.ds(row0, OROWS)], obuf_v)
        pltpu.sync_copy(cnt_sh.at[pl.ds(row0, OROWS)], cnt_v)

        def row_body(r, carry2):
            cv = cnt_v[r, pl.ds(0, 16)]
            inv = 1.0 / jnp.maximum(cv, 1.0)
            for j in range(H // 16):
                sl = pl.ds(j * 16, 16)
                obuf_v[r, sl] = obuf_v[r, sl] * inv
            return carry2
        lax.fori_loop(0, OROWS, row_body, 0, unroll=2)

        @pl.when(c == 0)
        def _():
            pltpu.sync_copy(obuf_v, aggA_out.at[pl.ds(row0, OROWS)])
        @pl.when(c == 1)
        def _():
            pltpu.sync_copy(obuf_v, aggB_out.at[pl.ds(row0, OROWS)])
        return carry
    lax.fori_loop(0, OSUB, out_piece, 0)


def _sc_call(pA, pB, nbr, dst, w2d, zrow, zcnt, ones_h):
    f32 = jnp.float32
    mesh = plsc.VectorSubcoreMesh(core_axis_name="c", subcore_axis_name="s")
    return pl.kernel(
        _sc_body,
        out_type=[
            jax.ShapeDtypeStruct((NPAD, H), f32),
            jax.ShapeDtypeStruct((NPAD, H), f32),
        ],
        mesh=mesh,
        scratch_types=[
            pltpu.VMEM((EPT,), jnp.int32),
            pltpu.VMEM((EPT,), f32),
            pltpu.VMEM((NCHUNK, K), jnp.int32),
            pltpu.VMEM((K, H), f32),
            pltpu.VMEM((K, 16), f32),
            pltpu.VMEM((OROWS, H), f32),
            pltpu.VMEM((OROWS, 16), f32),
            pltpu.VMEM_SHARED((NPAD, H), f32),
            pltpu.VMEM_SHARED((NPAD, 16), f32),
            pltpu.SemaphoreType.DMA,
        ],
        compiler_params=pltpu.CompilerParams(
            needs_layout_passes=False, use_tc_tiling_on_sc=False),
    )(pA, pB, nbr, dst, w2d, zrow, zcnt, ones_h)


@jax.jit
def kernel(node_reps, edges, edge_weights, prep_gamma, prep_beta, prep_mean,
           prep_var, prep_W, prep_b, upd_gamma, upd_beta, upd_mean, upd_var,
           upd_W, upd_b):
    f32 = jnp.float32
    s1 = prep_gamma / jnp.sqrt(prep_var + 1e-3)
    t1 = prep_beta - prep_mean * s1
    W1 = s1[:, None] * prep_W
    b1 = (t1 @ prep_W + prep_b)[None, :]
    s2 = upd_gamma / jnp.sqrt(upd_var + 1e-3)
    t2 = upd_beta - upd_mean * s2
    W2 = s2[:, None] * upd_W
    c2 = (t2 @ upd_W + upd_b)[None, :]
    WA = W2[:D]
    WB = [W2[D + q * H:D + (q + 1) * H] for q in range(4)]

    grid = N_NODES // BN
    ps = pl.pallas_call(
        _mm1_body,
        grid=(grid,),
        in_specs=[
            pl.BlockSpec((BN, D), lambda i: (i, 0)),
            pl.BlockSpec((D, D), lambda i: (0, 0)),
            pl.BlockSpec((1, D), lambda i: (0, 0)),
        ],
        out_specs=[pl.BlockSpec((BN, H), lambda i: (i, 0))] * 4,
        out_shape=[jax.ShapeDtypeStruct((N_NODES, H), f32)] * 4,
    )(node_reps, W1, b1)

    nbr = edges[1]
    dst = edges[0].reshape(NS, NCHUNK, K)
    w2d = edge_weights
    zrow = jnp.zeros((STRIPE, H), f32)
    zcnt = jnp.zeros((STRIPE, 16), f32)
    ones_h = jnp.ones((K, 16), f32)

    agg0, agg1 = _sc_call(ps[0], ps[1], nbr, dst, w2d, zrow, zcnt, ones_h)
    agg2, agg3 = _sc_call(ps[2], ps[3], nbr, dst, w2d, zrow, zcnt, ones_h)
    aggs = [a[:N_NODES] for a in (agg0, agg1, agg2, agg3)]

    out = pl.pallas_call(
        _mm2_body,
        grid=(grid,),
        in_specs=[
            pl.BlockSpec((BN, D), lambda i: (i, 0)),
            pl.BlockSpec((BN, H), lambda i: (i, 0)),
            pl.BlockSpec((BN, H), lambda i: (i, 0)),
            pl.BlockSpec((BN, H), lambda i: (i, 0)),
            pl.BlockSpec((BN, H), lambda i: (i, 0)),
            pl.BlockSpec((D, D), lambda i: (0, 0)),
            pl.BlockSpec((H, D), lambda i: (0, 0)),
            pl.BlockSpec((H, D), lambda i: (0, 0)),
            pl.BlockSpec((H, D), lambda i: (0, 0)),
            pl.BlockSpec((H, D), lambda i: (0, 0)),
            pl.BlockSpec((1, D), lambda i: (0, 0)),
        ],
        out_specs=pl.BlockSpec((BN, D), lambda i: (i, 0)),
        out_shape=jax.ShapeDtypeStruct((N_NODES, D), f32),
    )(node_reps, aggs[0], aggs[1], aggs[2], aggs[3],
      WA, WB[0], WB[1], WB[2], WB[3], c2)
    return out
